# baseline (speedup 1.0000x reference)
import functools

import jax
import jax.numpy as jnp
from jax import lax
from jax.experimental import pallas as pl
from jax.experimental.pallas import tpu as pltpu
from jax.experimental.pallas import tpu_sc as plsc

VOCAB = 1000000
EMBED = 256
BATCH = 4096
SEQ = 200

NW = 32
ROWS_PER_W = BATCH // NW
IDX_PER_W = ROWS_PER_W * SEQ
CHUNK = 40
NBUF = SEQ // CHUNK
NLANE = 16
NVEC = EMBED // NLANE

_mesh = plsc.VectorSubcoreMesh(core_axis_name="c", subcore_axis_name="s")


@functools.partial(
    pl.kernel,
    mesh=_mesh,
    out_type=jax.ShapeDtypeStruct((BATCH, EMBED), jnp.float32),
    scratch_types=(
        [pltpu.VMEM((IDX_PER_W,), jnp.int32)]
        + [pltpu.VMEM((CHUNK, EMBED), jnp.float32)] * NBUF
        + [pltpu.VMEM((8, EMBED), jnp.float32)]
        + [pltpu.SemaphoreType.DMA] * NBUF
    ),
)
def _pool(idx_hbm, emb_hbm, out_hbm, idx_v, *rest):
    bufs = rest[:NBUF]
    out_v = rest[NBUF]
    sems = rest[NBUF + 1:]
    wid = lax.axis_index("s") * 2 + lax.axis_index("c")
    base = wid * ROWS_PER_W

    pltpu.sync_copy(idx_hbm.at[pl.ds(wid * IDX_PER_W, IDX_PER_W)], idx_v)

    def chunk_src(g):
        return emb_hbm.at[idx_v.at[pl.ds(g * CHUNK, CHUNK)]]

    for k in range(NBUF):
        pltpu.make_async_copy(chunk_src(k), bufs[k], sems[k]).start()

    UNROLL = 8

    def accum_chunk(buf, accs):
        def body(j, a):
            for u in range(UNROLL):
                a = tuple(a[c] + buf[j * UNROLL + u, pl.ds(c * NLANE, NLANE)]
                          for c in range(NVEC))
            return a
        return lax.fori_loop(0, CHUNK // UNROLL, body, accs)

    def row_body(i, carry):
        accs = tuple(jnp.zeros((NLANE,), jnp.float32) for _ in range(NVEC))
        for k in range(NBUF):
            pltpu.make_async_copy(chunk_src(NBUF * i + k),
                                  bufs[k], sems[k]).wait()
            accs = accum_chunk(bufs[k], accs)

            @pl.when(i < ROWS_PER_W - 1)
            def _():
                pltpu.make_async_copy(chunk_src(NBUF * (i + 1) + k),
                                      bufs[k], sems[k]).start()

        slot = lax.rem(i, 8)
        for c in range(NVEC):
            out_v[slot, pl.ds(c * NLANE, NLANE)] = accs[c] * (1.0 / SEQ)

        @pl.when(slot == 7)
        def _():
            off = pl.multiple_of(base + i - 7, 8)
            pltpu.sync_copy(out_v, out_hbm.at[pl.ds(off, 8)])
        return carry

    lax.fori_loop(0, ROWS_PER_W, row_body, 0)


def _linear_body(x_ref, wt_ref, b_ref, o_ref):
    o_ref[...] = (
        jnp.dot(x_ref[...], wt_ref[...], preferred_element_type=jnp.float32)
        + b_ref[...]
    )


_linear = pl.pallas_call(
    _linear_body,
    out_shape=jax.ShapeDtypeStruct((BATCH, 2), jnp.float32),
)


def kernel(inputs, emb, W, b):
    idx_flat = inputs.astype(jnp.int32).reshape(NW * IDX_PER_W)
    pooled = _pool(idx_flat, emb)
    return _linear(pooled, W.T, b.reshape(1, 2))


# speedup vs baseline: 1.4830x; 1.4830x over previous
import functools

import jax
import jax.numpy as jnp
from jax import lax
from jax.experimental import pallas as pl
from jax.experimental.pallas import tpu as pltpu
from jax.experimental.pallas import tpu_sc as plsc

VOCAB = 1000000
EMBED = 256
BATCH = 4096
SEQ = 200

NW = 32
ROWS_PER_W = BATCH // NW
IDX_PER_W = ROWS_PER_W * SEQ
CHUNK = 40
NBUF = SEQ // CHUNK
NLANE = 16
NVEC = EMBED // NLANE

_mesh = plsc.VectorSubcoreMesh(core_axis_name="c", subcore_axis_name="s")


@functools.partial(
    pl.kernel,
    mesh=_mesh,
    out_type=jax.ShapeDtypeStruct((BATCH, EMBED), jnp.float32),
    scratch_types=(
        [pltpu.VMEM((IDX_PER_W,), jnp.int32)]
        + [pltpu.VMEM((CHUNK, EMBED), jnp.float32)] * NBUF
        + [pltpu.VMEM((8, EMBED), jnp.float32)]
        + [pltpu.SemaphoreType.DMA] * NBUF
    ),
)
def _pool(idx_hbm, emb_hbm, out_hbm, idx_v, *rest):
    bufs = rest[:NBUF]
    out_v = rest[NBUF]
    sems = rest[NBUF + 1:]
    wid = lax.axis_index("s") * 2 + lax.axis_index("c")
    base = wid * ROWS_PER_W

    pltpu.sync_copy(idx_hbm.at[pl.ds(wid * IDX_PER_W, IDX_PER_W)], idx_v)

    def chunk_src(g):
        return emb_hbm.at[idx_v.at[pl.ds(g * CHUNK, CHUNK)]]

    for k in range(NBUF):
        pltpu.make_async_copy(chunk_src(k), bufs[k], sems[k]).start()

    def accum_chunk(buf, accs):
        def body(j, a):
            return tuple(a[c] + buf[j, pl.ds(c * NLANE, NLANE)]
                         for c in range(NVEC))
        return plsc.parallel_loop(0, CHUNK, carry=accs, unroll=4)(body)

    def row_body(i, carry):
        accs = tuple(jnp.zeros((NLANE,), jnp.float32) for _ in range(NVEC))
        for k in range(NBUF):
            pltpu.make_async_copy(chunk_src(NBUF * i + k),
                                  bufs[k], sems[k]).wait()
            accs = accum_chunk(bufs[k], accs)

            @pl.when(i < ROWS_PER_W - 1)
            def _():
                pltpu.make_async_copy(chunk_src(NBUF * (i + 1) + k),
                                      bufs[k], sems[k]).start()

        slot = lax.rem(i, 8)
        for c in range(NVEC):
            out_v[slot, pl.ds(c * NLANE, NLANE)] = accs[c] * (1.0 / SEQ)

        @pl.when(slot == 7)
        def _():
            off = pl.multiple_of(base + i - 7, 8)
            pltpu.sync_copy(out_v, out_hbm.at[pl.ds(off, 8)])
        return carry

    lax.fori_loop(0, ROWS_PER_W, row_body, 0)


def _linear_body(x_ref, wt_ref, b_ref, o_ref):
    o_ref[...] = (
        jnp.dot(x_ref[...], wt_ref[...], preferred_element_type=jnp.float32)
        + b_ref[...]
    )


_linear = pl.pallas_call(
    _linear_body,
    out_shape=jax.ShapeDtypeStruct((BATCH, 2), jnp.float32),
)


def kernel(inputs, emb, W, b):
    idx_flat = inputs.astype(jnp.int32).reshape(NW * IDX_PER_W)
    pooled = _pool(idx_flat, emb)
    return _linear(pooled, W.T, b.reshape(1, 2))
